# baseline (speedup 1.0000x reference)
import jax
import jax.numpy as jnp
from jax.experimental import pallas as pl
from jax.experimental.pallas import tpu as pltpu

_B = 2048
_D = 512
_MARGIN = 0.2
_BM = 256
_NB = _B // _BM


def _normalize(v):
    return v * jax.lax.rsqrt(
        jnp.maximum(jnp.sum(v * v, axis=1, keepdims=True), 1e-24))


def _chunk_copy(x_hbm, xv, sem, t):
    return pltpu.make_async_copy(
        x_hbm.at[pl.ds(t * _BM, _BM), :],
        xv.at[pl.ds(t * _BM, _BM), :],
        sem)


def _body(x_hbm, out_ref, xv, abf, p_bf, sem):
    for t in range(_NB):
        _chunk_copy(x_hbm, xv, sem, t).start()

    eye = (jax.lax.broadcasted_iota(jnp.int32, (_BM, _BM), 0)
           == jax.lax.broadcasted_iota(jnp.int32, (_BM, _BM), 1))
    m = [None] * _NB
    rinv = [None] * _NB
    diag = [None] * _NB

    for t in range(_NB):
        _chunk_copy(x_hbm, xv, sem, t).wait()
        blk = xv[pl.ds(t * _BM, _BM), :]
        a = blk[:, 0:_D]
        p = blk[:, _D:2 * _D]
        p_bf[pl.ds(t * _BM, _BM), :] = _normalize(p).astype(jnp.bfloat16)
        abf[pl.ds(t * _BM, _BM), :] = a.astype(jnp.bfloat16)
        asq = jnp.sum(a * a, axis=1, keepdims=True)
        rinv[t] = jax.lax.rsqrt(jnp.maximum(asq, 1e-24))

        for k, j in [(t, jj) for jj in range(t)] + [(kk, t) for kk in range(t + 1)]:
            c = jax.lax.dot_general(
                abf[pl.ds(k * _BM, _BM), :],
                p_bf[pl.ds(j * _BM, _BM), :],
                (((1,), (1,)), ((), ())),
                preferred_element_type=jnp.float32,
            )
            if k == j:
                diag[k] = jnp.sum(jnp.where(eye, c, 0.0), axis=1, keepdims=True)
                c = jnp.where(eye, -jnp.inf, c)
            r = jnp.max(c, axis=1, keepdims=True)
            m[k] = r if m[k] is None else jnp.maximum(m[k], r)

    acc = jnp.zeros((1, 1), jnp.float32)
    for k in range(_NB):
        neg_sq = jnp.maximum(2.0 - 2.0 * m[k] * rinv[k], 0.0)
        pos_sq = jnp.maximum(2.0 - 2.0 * diag[k] * rinv[k], 0.0)
        acc = acc + jnp.sum(jnp.maximum(pos_sq - neg_sq + _MARGIN, 0.0),
                            axis=0, keepdims=True)
    out_ref[...] = acc * (1.0 / _B)


def kernel(x):
    out = pl.pallas_call(
        _body,
        in_specs=[pl.BlockSpec(memory_space=pl.ANY)],
        out_specs=pl.BlockSpec(memory_space=pltpu.VMEM),
        out_shape=jax.ShapeDtypeStruct((1, 1), jnp.float32),
        scratch_shapes=[
            pltpu.VMEM((_B, 2 * _D), jnp.float32),
            pltpu.VMEM((_B, _D), jnp.bfloat16),
            pltpu.VMEM((_B, _D), jnp.bfloat16),
            pltpu.SemaphoreType.DMA,
        ],
    )(x.reshape(_B, 2 * _D))
    return out[0, 0]


# speedup vs baseline: 1.0365x; 1.0365x over previous
import jax
import jax.numpy as jnp
from jax.experimental import pallas as pl
from jax.experimental.pallas import tpu as pltpu

_B = 2048
_D = 512
_MARGIN = 0.2
_BM = 256
_NB = _B // _BM


def _normalize(v):
    return v * jax.lax.rsqrt(
        jnp.maximum(jnp.sum(v * v, axis=1, keepdims=True), 1e-24))


def _chunk_copy(x_hbm, xv, sem, t):
    return pltpu.make_async_copy(
        x_hbm.at[pl.ds(t * _BM, _BM), :],
        xv.at[pl.ds(t * _BM, _BM), :],
        sem)


def _body(x_hbm, out_ref, xv, abf, p_bf, sem):
    for t in range(_NB):
        _chunk_copy(x_hbm, xv, sem, t).start()

    eye = (jax.lax.broadcasted_iota(jnp.int32, (_BM, _BM), 0)
           == jax.lax.broadcasted_iota(jnp.int32, (_BM, _BM), 1))

    rinv = [None] * _NB
    for t in range(_NB):
        _chunk_copy(x_hbm, xv, sem, t).wait()
        blk = xv[pl.ds(t * _BM, _BM), :]
        a = blk[:, 0:_D]
        p = blk[:, _D:2 * _D]
        p_bf[pl.ds(t * _BM, _BM), :] = _normalize(p).astype(jnp.bfloat16)
        abf[pl.ds(t * _BM, _BM), :] = a.astype(jnp.bfloat16)
        asq = jnp.sum(a * a, axis=1, keepdims=True)
        rinv[t] = jax.lax.rsqrt(jnp.maximum(asq, 1e-24))

    acc = jnp.zeros((1, 1), jnp.float32)
    for k in range(_NB):
        c = jax.lax.dot_general(
            abf[pl.ds(k * _BM, _BM), :], p_bf[...],
            (((1,), (1,)), ((), ())),
            preferred_element_type=jnp.float32,
        )
        dtile = c[:, k * _BM:(k + 1) * _BM]
        diag = jnp.sum(jnp.where(eye, dtile, 0.0),
                       axis=1, keepdims=True)
        folded = jnp.where(eye, -jnp.inf, dtile)
        for j in range(_NB):
            if j != k:
                folded = jnp.maximum(folded, c[:, j * _BM:(j + 1) * _BM])
        mx = jnp.max(folded, axis=1, keepdims=True)
        neg_sq = jnp.maximum(2.0 - 2.0 * mx * rinv[k], 0.0)
        pos_sq = jnp.maximum(2.0 - 2.0 * diag * rinv[k], 0.0)
        acc = acc + jnp.sum(jnp.maximum(pos_sq - neg_sq + _MARGIN, 0.0),
                            axis=0, keepdims=True)
    out_ref[...] = acc * (1.0 / _B)


def kernel(x):
    out = pl.pallas_call(
        _body,
        in_specs=[pl.BlockSpec(memory_space=pl.ANY)],
        out_specs=pl.BlockSpec(memory_space=pltpu.VMEM),
        out_shape=jax.ShapeDtypeStruct((1, 1), jnp.float32),
        scratch_shapes=[
            pltpu.VMEM((_B, 2 * _D), jnp.float32),
            pltpu.VMEM((_B, _D), jnp.bfloat16),
            pltpu.VMEM((_B, _D), jnp.bfloat16),
            pltpu.SemaphoreType.DMA,
        ],
    )(x.reshape(_B, 2 * _D))
    return out[0, 0]


# speedup vs baseline: 3.3106x; 3.1938x over previous
import jax
import jax.numpy as jnp
from jax.experimental import pallas as pl
from jax.experimental.pallas import tpu as pltpu

_B = 2048
_D = 512
_MARGIN = 0.2
_BM = 256
_NB = _B // _BM


def _normalize(v):
    return v * jax.lax.rsqrt(
        jnp.maximum(jnp.sum(v * v, axis=1, keepdims=True), 1e-24))


def _plane_chunk_copy(x_hbm, dst, sem, plane, t):
    return pltpu.make_async_copy(
        x_hbm.at[pl.ds(t * _BM, _BM), plane, :],
        dst.at[pl.ds(t * _BM, _BM), :],
        sem)


def _body(x_hbm, out_ref, a_f, p_f, p_bf, sem_a, sem_p):
    for t in range(_NB):
        _plane_chunk_copy(x_hbm, p_f, sem_p, 1, t).start()
    for t in range(_NB):
        _plane_chunk_copy(x_hbm, a_f, sem_a, 0, t).start()

    for t in range(_NB):
        _plane_chunk_copy(x_hbm, p_f, sem_p, 1, t).wait()
        p = p_f[pl.ds(t * _BM, _BM), :]
        p_bf[pl.ds(t * _BM, _BM), :] = _normalize(p).astype(jnp.bfloat16)

    eye = (jax.lax.broadcasted_iota(jnp.int32, (_BM, _BM), 0)
           == jax.lax.broadcasted_iota(jnp.int32, (_BM, _BM), 1))

    acc = jnp.zeros((1, 1), jnp.float32)
    for k in range(_NB):
        _plane_chunk_copy(x_hbm, a_f, sem_a, 0, k).wait()
        a = a_f[pl.ds(k * _BM, _BM), :]
        asq = jnp.sum(a * a, axis=1, keepdims=True)
        rinv = jax.lax.rsqrt(jnp.maximum(asq, 1e-24))
        c = jax.lax.dot_general(
            a.astype(jnp.bfloat16), p_bf[...],
            (((1,), (1,)), ((), ())),
            preferred_element_type=jnp.float32,
        )
        dtile = c[:, k * _BM:(k + 1) * _BM]
        diag = jnp.sum(jnp.where(eye, dtile, 0.0),
                       axis=1, keepdims=True)
        folded = jnp.where(eye, -jnp.inf, dtile)
        for j in range(_NB):
            if j != k:
                folded = jnp.maximum(folded, c[:, j * _BM:(j + 1) * _BM])
        mx = jnp.max(folded, axis=1, keepdims=True)
        neg_sq = jnp.maximum(2.0 - 2.0 * mx * rinv, 0.0)
        pos_sq = jnp.maximum(2.0 - 2.0 * diag * rinv, 0.0)
        acc = acc + jnp.sum(jnp.maximum(pos_sq - neg_sq + _MARGIN, 0.0),
                            axis=0, keepdims=True)
    out_ref[...] = acc * (1.0 / _B)


def kernel(x):
    out = pl.pallas_call(
        _body,
        in_specs=[pl.BlockSpec(memory_space=pl.ANY)],
        out_specs=pl.BlockSpec(memory_space=pltpu.VMEM),
        out_shape=jax.ShapeDtypeStruct((1, 1), jnp.float32),
        scratch_shapes=[
            pltpu.VMEM((_B, _D), jnp.float32),
            pltpu.VMEM((_B, _D), jnp.float32),
            pltpu.VMEM((_B, _D), jnp.bfloat16),
            pltpu.SemaphoreType.DMA,
            pltpu.SemaphoreType.DMA,
        ],
    )(x)
    return out[0, 0]


# speedup vs baseline: 3.8525x; 1.1637x over previous
import jax
import jax.numpy as jnp
from jax.experimental import pallas as pl
from jax.experimental.pallas import tpu as pltpu

_B = 2048
_D = 512
_MARGIN = 0.2
_BM = 512
_NB = _B // _BM


def _normalize(v):
    return v * jax.lax.rsqrt(
        jnp.maximum(jnp.sum(v * v, axis=1, keepdims=True), 1e-24))


def _plane_chunk_copy(x_hbm, dst, sem, plane, t):
    return pltpu.make_async_copy(
        x_hbm.at[pl.ds(t * _BM, _BM), plane, :],
        dst.at[pl.ds(t * _BM, _BM), :],
        sem)


def _body(x_hbm, out_ref, a_f, p_f, p_bf, sem_a, sem_p):
    for t in range(_NB):
        _plane_chunk_copy(x_hbm, p_f, sem_p, 1, t).start()
    for t in range(_NB):
        _plane_chunk_copy(x_hbm, a_f, sem_a, 0, t).start()

    for t in range(_NB):
        _plane_chunk_copy(x_hbm, p_f, sem_p, 1, t).wait()
        p = p_f[pl.ds(t * _BM, _BM), :]
        p_bf[pl.ds(t * _BM, _BM), :] = _normalize(p).astype(jnp.bfloat16)

    eye = (jax.lax.broadcasted_iota(jnp.int32, (_BM, _BM), 0)
           == jax.lax.broadcasted_iota(jnp.int32, (_BM, _BM), 1))

    acc = jnp.zeros((1, 1), jnp.float32)
    for k in range(_NB):
        _plane_chunk_copy(x_hbm, a_f, sem_a, 0, k).wait()
        a = a_f[pl.ds(k * _BM, _BM), :]
        asq = jnp.sum(a * a, axis=1, keepdims=True)
        rinv = jax.lax.rsqrt(jnp.maximum(asq, 1e-24))
        c = jax.lax.dot_general(
            a.astype(jnp.bfloat16), p_bf[...],
            (((1,), (1,)), ((), ())),
            preferred_element_type=jnp.float32,
        )
        dtile = c[:, k * _BM:(k + 1) * _BM]
        diag = jnp.sum(jnp.where(eye, dtile, 0.0),
                       axis=1, keepdims=True)
        folded = jnp.where(eye, -jnp.inf, dtile)
        for j in range(_NB):
            if j != k:
                folded = jnp.maximum(folded, c[:, j * _BM:(j + 1) * _BM])
        mx = jnp.max(folded, axis=1, keepdims=True)
        neg_sq = jnp.maximum(2.0 - 2.0 * mx * rinv, 0.0)
        pos_sq = jnp.maximum(2.0 - 2.0 * diag * rinv, 0.0)
        acc = acc + jnp.sum(jnp.maximum(pos_sq - neg_sq + _MARGIN, 0.0),
                            axis=0, keepdims=True)
    out_ref[...] = acc * (1.0 / _B)


def kernel(x):
    out = pl.pallas_call(
        _body,
        in_specs=[pl.BlockSpec(memory_space=pl.ANY)],
        out_specs=pl.BlockSpec(memory_space=pltpu.VMEM),
        out_shape=jax.ShapeDtypeStruct((1, 1), jnp.float32),
        scratch_shapes=[
            pltpu.VMEM((_B, _D), jnp.float32),
            pltpu.VMEM((_B, _D), jnp.float32),
            pltpu.VMEM((_B, _D), jnp.bfloat16),
            pltpu.SemaphoreType.DMA,
            pltpu.SemaphoreType.DMA,
        ],
    )(x)
    return out[0, 0]


# speedup vs baseline: 4.0898x; 1.0616x over previous
import jax
import jax.numpy as jnp
from jax.experimental import pallas as pl
from jax.experimental.pallas import tpu as pltpu

_B = 2048
_D = 512
_MARGIN = 0.2
_BM = 1024
_NB = _B // _BM


def _normalize(v):
    return v * jax.lax.rsqrt(
        jnp.maximum(jnp.sum(v * v, axis=1, keepdims=True), 1e-24))


def _plane_chunk_copy(x_hbm, dst, sem, plane, t):
    return pltpu.make_async_copy(
        x_hbm.at[pl.ds(t * _BM, _BM), plane, :],
        dst.at[pl.ds(t * _BM, _BM), :],
        sem)


def _body(x_hbm, out_ref, a_f, p_f, p_bf, sem_a, sem_p):
    for t in range(_NB):
        _plane_chunk_copy(x_hbm, p_f, sem_p, 1, t).start()
    for t in range(_NB):
        _plane_chunk_copy(x_hbm, a_f, sem_a, 0, t).start()

    for t in range(_NB):
        _plane_chunk_copy(x_hbm, p_f, sem_p, 1, t).wait()
        p = p_f[pl.ds(t * _BM, _BM), :]
        p_bf[pl.ds(t * _BM, _BM), :] = _normalize(p).astype(jnp.bfloat16)

    eye = (jax.lax.broadcasted_iota(jnp.int32, (_BM, _BM), 0)
           == jax.lax.broadcasted_iota(jnp.int32, (_BM, _BM), 1))

    acc = jnp.zeros((1, 1), jnp.float32)
    for k in range(_NB):
        _plane_chunk_copy(x_hbm, a_f, sem_a, 0, k).wait()
        a = a_f[pl.ds(k * _BM, _BM), :]
        asq = jnp.sum(a * a, axis=1, keepdims=True)
        rinv = jax.lax.rsqrt(jnp.maximum(asq, 1e-24))
        c = jax.lax.dot_general(
            a.astype(jnp.bfloat16), p_bf[...],
            (((1,), (1,)), ((), ())),
            preferred_element_type=jnp.float32,
        )
        dtile = c[:, k * _BM:(k + 1) * _BM]
        diag = jnp.sum(jnp.where(eye, dtile, 0.0),
                       axis=1, keepdims=True)
        folded = jnp.where(eye, -jnp.inf, dtile)
        for j in range(_NB):
            if j != k:
                folded = jnp.maximum(folded, c[:, j * _BM:(j + 1) * _BM])
        mx = jnp.max(folded, axis=1, keepdims=True)
        neg_sq = jnp.maximum(2.0 - 2.0 * mx * rinv, 0.0)
        pos_sq = jnp.maximum(2.0 - 2.0 * diag * rinv, 0.0)
        acc = acc + jnp.sum(jnp.maximum(pos_sq - neg_sq + _MARGIN, 0.0),
                            axis=0, keepdims=True)
    out_ref[...] = acc * (1.0 / _B)


def kernel(x):
    out = pl.pallas_call(
        _body,
        in_specs=[pl.BlockSpec(memory_space=pl.ANY)],
        out_specs=pl.BlockSpec(memory_space=pltpu.VMEM),
        out_shape=jax.ShapeDtypeStruct((1, 1), jnp.float32),
        scratch_shapes=[
            pltpu.VMEM((_B, _D), jnp.float32),
            pltpu.VMEM((_B, _D), jnp.float32),
            pltpu.VMEM((_B, _D), jnp.bfloat16),
            pltpu.SemaphoreType.DMA,
            pltpu.SemaphoreType.DMA,
        ],
    )(x)
    return out[0, 0]
